# baseline (speedup 1.0000x reference)
import functools

import jax
import jax.numpy as jnp
from jax.experimental import pallas as pl
from jax.experimental.pallas import tpu as pltpu

N = 10000
E = 160000
D_IN = 256
D_EDGE = 16
H = 256
OUT = 256
L = 3

BN = 1000
BE = 2000


def _ln(x, g, b, eps=1e-5):
    mu = jnp.mean(x, axis=-1, keepdims=True)
    var = jnp.mean((x - mu) ** 2, axis=-1, keepdims=True)
    return (x - mu) / jnp.sqrt(var + eps) * g + b


def _node_enc_kernel(x_ref, w_ref, b_ref, g_ref, bb_ref, h_ref, t_ref):
    h = jnp.dot(x_ref[...], w_ref[...], preferred_element_type=jnp.float32)
    h = h + b_ref[...]
    h_ref[...] = h
    t = jax.nn.relu(_ln(h, g_ref[...], bb_ref[...]))
    t_ref[0] = t[:, :128]
    t_ref[1] = t[:, 128:]


def _node_encoder(x, w, b, g, bb):
    nblk = N // BN
    return pl.pallas_call(
        _node_enc_kernel,
        grid=(nblk,),
        in_specs=[
            pl.BlockSpec((BN, D_IN), lambda i: (i, 0)),
            pl.BlockSpec((D_IN, H), lambda i: (0, 0)),
            pl.BlockSpec((1, H), lambda i: (0, 0)),
            pl.BlockSpec((1, H), lambda i: (0, 0)),
            pl.BlockSpec((1, H), lambda i: (0, 0)),
        ],
        out_specs=[
            pl.BlockSpec((BN, H), lambda i: (i, 0)),
            pl.BlockSpec((2, BN, 128), lambda i: (0, i, 0)),
        ],
        out_shape=[
            jax.ShapeDtypeStruct((N, H), jnp.float32),
            jax.ShapeDtypeStruct((2, N, 128), jnp.float32),
        ],
    )(x, w, b.reshape(1, H), g.reshape(1, H), bb.reshape(1, H))


def _edge_enc_kernel(a_ref, w_ref, b_ref, o_ref):
    ea = jnp.dot(a_ref[...], w_ref[...], preferred_element_type=jnp.float32)
    ea = ea + b_ref[...]
    o_ref[0] = ea[:, :128]
    o_ref[1] = ea[:, 128:]


def _edge_encoder(a, w, b):
    nblk = E // BE
    return pl.pallas_call(
        _edge_enc_kernel,
        grid=(nblk,),
        in_specs=[
            pl.BlockSpec((BE, D_EDGE), lambda i: (i, 0)),
            pl.BlockSpec((D_EDGE, H), lambda i: (0, 0)),
            pl.BlockSpec((1, H), lambda i: (0, 0)),
        ],
        out_specs=pl.BlockSpec((2, BE, 128), lambda i: (0, i, 0)),
        out_shape=jax.ShapeDtypeStruct((2, E, 128), jnp.float32),
    )(a, w, b.reshape(1, H))


def _mlp_mid_kernel(ss_ref, dinv_ref, eps_ref, t_ref, h_ref,
                    w1_ref, b1_ref, g1_ref, bb1_ref, w2_ref, b2_ref,
                    gn_ref, bn_ref, h_out, t_out):
    aggr = jnp.concatenate([ss_ref[0], ss_ref[1]], axis=-1)
    aggr = aggr * dinv_ref[...] + eps_ref[...]
    t = jnp.concatenate([t_ref[0], t_ref[1]], axis=-1)
    out = aggr + t
    m = jnp.dot(out, w1_ref[...], preferred_element_type=jnp.float32) + b1_ref[...]
    m = jax.nn.relu(_ln(m, g1_ref[...], bb1_ref[...]))
    m = jnp.dot(m, w2_ref[...], preferred_element_type=jnp.float32) + b2_ref[...]
    h_new = h_ref[...] + m
    h_out[...] = h_new
    tn = jax.nn.relu(_ln(h_new, gn_ref[...], bn_ref[...]))
    t_out[0] = tn[:, :128]
    t_out[1] = tn[:, 128:]


def _mlp_last_kernel(ss_ref, dinv_ref, eps_ref, t_ref, h_ref,
                     w1_ref, b1_ref, g1_ref, bb1_ref, w2_ref, b2_ref,
                     gn_ref, bn_ref, wo_ref, bo_ref, y_out):
    aggr = jnp.concatenate([ss_ref[0], ss_ref[1]], axis=-1)
    aggr = aggr * dinv_ref[...] + eps_ref[...]
    t = jnp.concatenate([t_ref[0], t_ref[1]], axis=-1)
    out = aggr + t
    m = jnp.dot(out, w1_ref[...], preferred_element_type=jnp.float32) + b1_ref[...]
    m = jax.nn.relu(_ln(m, g1_ref[...], bb1_ref[...]))
    m = jnp.dot(m, w2_ref[...], preferred_element_type=jnp.float32) + b2_ref[...]
    h_new = h_ref[...] + m
    y = jax.nn.relu(_ln(h_new, gn_ref[...], bn_ref[...]))
    y_out[...] = jnp.dot(y, wo_ref[...], preferred_element_type=jnp.float32) + bo_ref[...]


def _mlp_block(ss, dinv, eps_row, t, h, w1, b1, g1, bb1, w2, b2, gn, bn,
               wo=None, bo=None):
    nblk = N // BN
    in_specs = [
        pl.BlockSpec((2, BN, 128), lambda i: (0, i, 0)),
        pl.BlockSpec((BN, 1), lambda i: (i, 0)),
        pl.BlockSpec((BN, 1), lambda i: (i, 0)),
        pl.BlockSpec((2, BN, 128), lambda i: (0, i, 0)),
        pl.BlockSpec((BN, H), lambda i: (i, 0)),
        pl.BlockSpec((H, 2 * H), lambda i: (0, 0)),
        pl.BlockSpec((1, 2 * H), lambda i: (0, 0)),
        pl.BlockSpec((1, 2 * H), lambda i: (0, 0)),
        pl.BlockSpec((1, 2 * H), lambda i: (0, 0)),
        pl.BlockSpec((2 * H, H), lambda i: (0, 0)),
        pl.BlockSpec((1, H), lambda i: (0, 0)),
        pl.BlockSpec((1, H), lambda i: (0, 0)),
        pl.BlockSpec((1, H), lambda i: (0, 0)),
    ]
    args = [ss, dinv, eps_row, t, h, w1, b1.reshape(1, -1), g1.reshape(1, -1),
            bb1.reshape(1, -1), w2, b2.reshape(1, -1), gn.reshape(1, -1),
            bn.reshape(1, -1)]
    if wo is None:
        return pl.pallas_call(
            _mlp_mid_kernel,
            grid=(nblk,),
            in_specs=in_specs,
            out_specs=[
                pl.BlockSpec((BN, H), lambda i: (i, 0)),
                pl.BlockSpec((2, BN, 128), lambda i: (0, i, 0)),
            ],
            out_shape=[
                jax.ShapeDtypeStruct((N, H), jnp.float32),
                jax.ShapeDtypeStruct((2, N, 128), jnp.float32),
            ],
        )(*args)
    in_specs += [
        pl.BlockSpec((H, OUT), lambda i: (0, 0)),
        pl.BlockSpec((1, OUT), lambda i: (0, 0)),
    ]
    args += [wo, bo.reshape(1, OUT)]
    return pl.pallas_call(
        _mlp_last_kernel,
        grid=(nblk,),
        in_specs=in_specs,
        out_specs=pl.BlockSpec((BN, OUT), lambda i: (i, 0)),
        out_shape=jax.ShapeDtypeStruct((N, OUT), jnp.float32),
    )(*args)


def _edge_pass(t, ea, src, dst):
    tf = jnp.concatenate([t[0], t[1]], axis=-1)
    ef = jnp.concatenate([ea[0], ea[1]], axis=-1)
    msg = jax.nn.relu(tf[src] + ef)
    ss = jax.ops.segment_sum(msg, dst, num_segments=N)
    return jnp.stack([ss[:, :128], ss[:, 128:]], axis=0)


def _degree(dst):
    return jax.ops.segment_sum(jnp.ones((E,), jnp.float32), dst, num_segments=N)


def kernel(x, edge_index, edge_attr, W_node, b_node, W_edge, b_edge,
           ln_g, ln_b, W1, b1, lng1, lnb1, W2, b2,
           gamma_out, beta_out, W_out, b_out):
    src = edge_index[0]
    dst = edge_index[1]

    h, t = _node_encoder(x, W_node, b_node, ln_g[0], ln_b[0])
    ea = _edge_encoder(edge_attr, W_edge, b_edge)

    cnt = _degree(dst)
    dinv = (1.0 / jnp.maximum(cnt, 1.0)).reshape(N, 1)
    eps_row = jnp.where(cnt > 0.0, jnp.float32(1e-7), 0.0).reshape(N, 1)

    for i in range(L):
        ss = _edge_pass(t, ea, src, dst)
        if i < L - 1:
            h, t = _mlp_block(ss, dinv, eps_row, t, h, W1[i], b1[i], lng1[i],
                              lnb1[i], W2[i], b2[i], ln_g[i + 1], ln_b[i + 1])
        else:
            y = _mlp_block(ss, dinv, eps_row, t, h, W1[i], b1[i], lng1[i],
                           lnb1[i], W2[i], b2[i], gamma_out, beta_out,
                           W_out, b_out)
    return y


# speedup vs baseline: 2.1248x; 2.1248x over previous
import dataclasses
import functools

import jax
import jax.numpy as jnp
from jax import lax
from jax.experimental import pallas as pl
from jax.experimental.pallas import tpu as pltpu
from jax.experimental.pallas import tpu_sc as plsc

N = 10000
E = 160000
D_IN = 256
D_EDGE = 16
H = 256
OUT = 256
L = 3

BN = 1000
BE = 2000


def _ln(x, g, b, eps=1e-5):
    mu = jnp.mean(x, axis=-1, keepdims=True)
    var = jnp.mean((x - mu) ** 2, axis=-1, keepdims=True)
    return (x - mu) / jnp.sqrt(var + eps) * g + b


def _node_enc_kernel(x_ref, w_ref, b_ref, g_ref, bb_ref, h_ref, t_ref):
    h = jnp.dot(x_ref[...], w_ref[...], preferred_element_type=jnp.float32)
    h = h + b_ref[...]
    h_ref[...] = h
    t = jax.nn.relu(_ln(h, g_ref[...], bb_ref[...]))
    t_ref[0] = t[:, :128]
    t_ref[1] = t[:, 128:]


def _node_encoder(x, w, b, g, bb):
    nblk = N // BN
    return pl.pallas_call(
        _node_enc_kernel,
        grid=(nblk,),
        in_specs=[
            pl.BlockSpec((BN, D_IN), lambda i: (i, 0)),
            pl.BlockSpec((D_IN, H), lambda i: (0, 0)),
            pl.BlockSpec((1, H), lambda i: (0, 0)),
            pl.BlockSpec((1, H), lambda i: (0, 0)),
            pl.BlockSpec((1, H), lambda i: (0, 0)),
        ],
        out_specs=[
            pl.BlockSpec((BN, H), lambda i: (i, 0)),
            pl.BlockSpec((2, BN, 128), lambda i: (0, i, 0)),
        ],
        out_shape=[
            jax.ShapeDtypeStruct((N, H), jnp.float32),
            jax.ShapeDtypeStruct((2, N, 128), jnp.float32),
        ],
    )(x, w, b.reshape(1, H), g.reshape(1, H), bb.reshape(1, H))


def _edge_enc_kernel(a_ref, w_ref, b_ref, o_ref):
    ea = jnp.dot(a_ref[...], w_ref[...], preferred_element_type=jnp.float32)
    ea = ea + b_ref[...]
    o_ref[0] = ea[:, :128]
    o_ref[1] = ea[:, 128:]


def _edge_encoder(a, w, b):
    nblk = E // BE
    return pl.pallas_call(
        _edge_enc_kernel,
        grid=(nblk,),
        in_specs=[
            pl.BlockSpec((BE, D_EDGE), lambda i: (i, 0)),
            pl.BlockSpec((D_EDGE, H), lambda i: (0, 0)),
            pl.BlockSpec((1, H), lambda i: (0, 0)),
        ],
        out_specs=pl.BlockSpec((2, BE, 128), lambda i: (0, i, 0)),
        out_shape=jax.ShapeDtypeStruct((2, E, 128), jnp.float32),
    )(a, w, b.reshape(1, H))


def _mlp_mid_kernel(ss_ref, dinv_ref, eps_ref, t_ref, h_ref,
                    w1_ref, b1_ref, g1_ref, bb1_ref, w2_ref, b2_ref,
                    gn_ref, bn_ref, h_out, t_out):
    aggr = jnp.concatenate([ss_ref[0], ss_ref[1]], axis=-1)
    aggr = aggr * dinv_ref[...] + eps_ref[...]
    t = jnp.concatenate([t_ref[0], t_ref[1]], axis=-1)
    out = aggr + t
    m = jnp.dot(out, w1_ref[...], preferred_element_type=jnp.float32) + b1_ref[...]
    m = jax.nn.relu(_ln(m, g1_ref[...], bb1_ref[...]))
    m = jnp.dot(m, w2_ref[...], preferred_element_type=jnp.float32) + b2_ref[...]
    h_new = h_ref[...] + m
    h_out[...] = h_new
    tn = jax.nn.relu(_ln(h_new, gn_ref[...], bn_ref[...]))
    t_out[0] = tn[:, :128]
    t_out[1] = tn[:, 128:]


def _mlp_last_kernel(ss_ref, dinv_ref, eps_ref, t_ref, h_ref,
                     w1_ref, b1_ref, g1_ref, bb1_ref, w2_ref, b2_ref,
                     gn_ref, bn_ref, wo_ref, bo_ref, y_out):
    aggr = jnp.concatenate([ss_ref[0], ss_ref[1]], axis=-1)
    aggr = aggr * dinv_ref[...] + eps_ref[...]
    t = jnp.concatenate([t_ref[0], t_ref[1]], axis=-1)
    out = aggr + t
    m = jnp.dot(out, w1_ref[...], preferred_element_type=jnp.float32) + b1_ref[...]
    m = jax.nn.relu(_ln(m, g1_ref[...], bb1_ref[...]))
    m = jnp.dot(m, w2_ref[...], preferred_element_type=jnp.float32) + b2_ref[...]
    h_new = h_ref[...] + m
    y = jax.nn.relu(_ln(h_new, gn_ref[...], bn_ref[...]))
    y_out[...] = jnp.dot(y, wo_ref[...], preferred_element_type=jnp.float32) + bo_ref[...]


def _mlp_block(ss, dinv, eps_row, t, h, w1, b1, g1, bb1, w2, b2, gn, bn,
               wo=None, bo=None):
    nblk = N // BN
    in_specs = [
        pl.BlockSpec((2, BN, 128), lambda i: (0, i, 0)),
        pl.BlockSpec((BN, 1), lambda i: (i, 0)),
        pl.BlockSpec((BN, 1), lambda i: (i, 0)),
        pl.BlockSpec((2, BN, 128), lambda i: (0, i, 0)),
        pl.BlockSpec((BN, H), lambda i: (i, 0)),
        pl.BlockSpec((H, 2 * H), lambda i: (0, 0)),
        pl.BlockSpec((1, 2 * H), lambda i: (0, 0)),
        pl.BlockSpec((1, 2 * H), lambda i: (0, 0)),
        pl.BlockSpec((1, 2 * H), lambda i: (0, 0)),
        pl.BlockSpec((2 * H, H), lambda i: (0, 0)),
        pl.BlockSpec((1, H), lambda i: (0, 0)),
        pl.BlockSpec((1, H), lambda i: (0, 0)),
        pl.BlockSpec((1, H), lambda i: (0, 0)),
    ]
    args = [ss, dinv, eps_row, t, h, w1, b1.reshape(1, -1), g1.reshape(1, -1),
            bb1.reshape(1, -1), w2, b2.reshape(1, -1), gn.reshape(1, -1),
            bn.reshape(1, -1)]
    if wo is None:
        return pl.pallas_call(
            _mlp_mid_kernel,
            grid=(nblk,),
            in_specs=in_specs,
            out_specs=[
                pl.BlockSpec((BN, H), lambda i: (i, 0)),
                pl.BlockSpec((2, BN, 128), lambda i: (0, i, 0)),
            ],
            out_shape=[
                jax.ShapeDtypeStruct((N, H), jnp.float32),
                jax.ShapeDtypeStruct((2, N, 128), jnp.float32),
            ],
        )(*args)
    in_specs += [
        pl.BlockSpec((H, OUT), lambda i: (0, 0)),
        pl.BlockSpec((1, OUT), lambda i: (0, 0)),
    ]
    args += [wo, bo.reshape(1, OUT)]
    return pl.pallas_call(
        _mlp_last_kernel,
        grid=(nblk,),
        in_specs=in_specs,
        out_specs=pl.BlockSpec((BN, OUT), lambda i: (i, 0)),
        out_shape=jax.ShapeDtypeStruct((N, OUT), jnp.float32),
    )(*args)


_SC_MESH = plsc.VectorSubcoreMesh(core_axis_name="c", subcore_axis_name="s")
NSUB = 16
C = 80
EPW = E // NSUB
NCHUNK = EPW // C
NB = 4
NROW = E // C
NP = 10240

_SC_PARAMS = pltpu.CompilerParams()
if "needs_layout_passes" in pltpu.CompilerParams.__dataclass_fields__:
    _SC_PARAMS = dataclasses.replace(_SC_PARAMS, needs_layout_passes=False)


def _edge_sc_body(t_hbm, ea_hbm, srcadj_hbm, dst_hbm, out_hbm,
                  acc, rows, eab, sidx, didx,
                  sem_si, sem_di, sem_ea, sem_g, sem_sc):
    c = lax.axis_index("c")
    s = lax.axis_index("s")
    src_off = c * E + s * EPW
    dst_off = s * EPW
    ea_row0 = c * E + s * EPW
    acc_row0 = s * (NP // NSUB)

    @pl.loop(0, C)
    def _(r):
        for col in range(0, 128, 16):
            eab[0, r, pl.ds(col, 16)] = jnp.zeros((16,), jnp.float32)

    @pl.loop(0, (NP // NSUB) // C)
    def _(k):
        pltpu.sync_copy(eab.at[0], acc.at[pl.ds(acc_row0 + k * C, C)])

    plsc.subcore_barrier()

    def fetch(g):
        b = lax.rem(g, 2)
        bd = lax.rem(g, NB)
        pltpu.async_copy(srcadj_hbm.at[pl.ds(src_off + g * C, C)], sidx.at[b],
                         sem_si.at[b])
        pltpu.async_copy(dst_hbm.at[pl.ds(dst_off + g * C, C)], didx.at[bd],
                         sem_di.at[bd])
        pltpu.async_copy(ea_hbm.at[pl.ds(ea_row0 + g * C, C)], eab.at[b],
                         sem_ea.at[b])

    def drain_scatter(g):
        b = lax.rem(g, 2)
        bd = lax.rem(g, NB)
        pltpu.make_async_copy(rows.at[b], acc.at[didx.at[bd]],
                              sem_sc.at[b]).wait()

    def fire(g):
        b = lax.rem(g, 2)
        pltpu.make_async_copy(srcadj_hbm.at[pl.ds(src_off + g * C, C)],
                              sidx.at[b], sem_si.at[b]).wait()
        pltpu.async_copy(t_hbm.at[sidx.at[b]], rows.at[b], sem_g.at[b])

    def compute(g):
        b = lax.rem(g, 2)
        bd = lax.rem(g, NB)
        pltpu.make_async_copy(dst_hbm.at[pl.ds(dst_off + g * C, C)],
                              didx.at[bd], sem_di.at[bd]).wait()
        pltpu.make_async_copy(ea_hbm.at[pl.ds(ea_row0 + g * C, C)], eab.at[b],
                              sem_ea.at[b]).wait()
        pltpu.make_async_copy(t_hbm.at[sidx.at[b]], rows.at[b],
                              sem_g.at[b]).wait()

        @pl.loop(0, C)
        def _(r):
            for col in range(0, 128, 16):
                v = rows[b, r, pl.ds(col, 16)] + eab[b, r, pl.ds(col, 16)]
                rows[b, r, pl.ds(col, 16)] = jnp.maximum(v, 0.0)

        pltpu.async_copy(rows.at[b], acc.at[didx.at[bd]], sem_sc.at[b],
                         add=True)

    fetch(0)
    fetch(1)
    fire(0)

    @pl.loop(0, NCHUNK)
    def _(g):
        @pl.when(g + 1 < NCHUNK)
        def _():
            @pl.when(g >= 2)
            def _():
                drain_scatter(g - 1)
            fire(g + 1)

        compute(g)

        @pl.when(g + 2 < NCHUNK)
        def _():
            fetch(g + 2)

    drain_scatter(NCHUNK - 2)
    drain_scatter(NCHUNK - 1)

    plsc.subcore_barrier()
    pltpu.sync_copy(acc.at[pl.ds(acc_row0, NP // NSUB)],
                    out_hbm.at[pl.ds(c * NP + acc_row0, NP // NSUB)])


@jax.jit
def _edge_pass_sc(t_flat, ea_flat, srcadj, dst1):
    f = pl.kernel(
        _edge_sc_body,
        out_type=jax.ShapeDtypeStruct((2 * NP, 128), jnp.float32),
        mesh=_SC_MESH,
        scratch_types=[
            pltpu.VMEM_SHARED((NP, 128), jnp.float32),
            pltpu.VMEM((2, C, 128), jnp.float32),
            pltpu.VMEM((2, C, 128), jnp.float32),
            pltpu.VMEM((2, C), jnp.int32),
            pltpu.VMEM((NB, C), jnp.int32),
            pltpu.SemaphoreType.DMA((2,)),
            pltpu.SemaphoreType.DMA((NB,)),
            pltpu.SemaphoreType.DMA((2,)),
            pltpu.SemaphoreType.DMA((2,)),
            pltpu.SemaphoreType.DMA((2,)),
        ],
        compiler_params=_SC_PARAMS,
    )
    return f(t_flat, ea_flat, srcadj, dst1)


NPAD = 10240
EDEG = E // 32


def _deg_sc_body(dst_hbm, out_hbm, hist, dbuf, sbuf, shared, sem):
    c = lax.axis_index("c")
    s = lax.axis_index("s")
    w = c * NSUB + s

    @pl.loop(0, NPAD, step=16)
    def _(i):
        hist[pl.ds(i, 16)] = jnp.zeros((16,), jnp.float32)

    pltpu.sync_copy(dst_hbm.at[pl.ds(w * EDEG, EDEG)], dbuf)
    ones = jnp.ones((16,), jnp.float32)

    @pl.loop(0, EDEG - 16, step=16)
    def _(i):
        plsc.addupdate_scatter(hist, [dbuf[pl.ds(i, 16)]], ones)

    tail_mask = lax.iota(jnp.int32, 16) >= 8
    plsc.addupdate_scatter(hist, [dbuf[pl.ds(EDEG - 16, 16)]], ones,
                           mask=tail_mask)

    pltpu.sync_copy(hist, shared.at[s])
    plsc.subcore_barrier()

    col0 = s * (NPAD // NSUB)
    pltpu.sync_copy(shared.at[:, pl.ds(col0, NPAD // NSUB)], sbuf)

    @pl.loop(0, NPAD // NSUB, step=16)
    def _(i):
        v = sbuf[0, pl.ds(i, 16)]
        for r in range(1, NSUB):
            v = v + sbuf[r, pl.ds(i, 16)]
        hist[pl.ds(i, 16)] = v

    pltpu.sync_copy(hist.at[pl.ds(0, NPAD // NSUB)],
                    out_hbm.at[c].at[pl.ds(col0, NPAD // NSUB)])


@jax.jit
def _degree_sc(dst):
    f = pl.kernel(
        _deg_sc_body,
        out_type=jax.ShapeDtypeStruct((2, NPAD), jnp.float32),
        mesh=_SC_MESH,
        scratch_types=[
            pltpu.VMEM((NPAD,), jnp.float32),
            pltpu.VMEM((EDEG,), jnp.int32),
            pltpu.VMEM((NSUB, NPAD // NSUB), jnp.float32),
            pltpu.VMEM_SHARED((NSUB, NPAD), jnp.float32),
            pltpu.SemaphoreType.DMA,
        ],
        compiler_params=_SC_PARAMS,
    )
    return f(dst)


def kernel(x, edge_index, edge_attr, W_node, b_node, W_edge, b_edge,
           ln_g, ln_b, W1, b1, lng1, lnb1, W2, b2,
           gamma_out, beta_out, W_out, b_out):
    src = edge_index[0]
    dst = edge_index[1]
    srcadj = jnp.concatenate([src, src + N], axis=0)

    h, t = _node_encoder(x, W_node, b_node, ln_g[0], ln_b[0])
    ea = _edge_encoder(edge_attr, W_edge, b_edge)
    ea_flat = ea.reshape(2 * E, 128)

    cntp = _degree_sc(dst)
    cnt = (cntp[0] + cntp[1])[:N]
    dinv = (1.0 / jnp.maximum(cnt, 1.0)).reshape(N, 1)
    eps_row = jnp.where(cnt > 0.0, jnp.float32(1e-7), 0.0).reshape(N, 1)

    for i in range(L):
        ssf = _edge_pass_sc(t.reshape(2 * N, 128), ea_flat, srcadj, dst)
        ss = jnp.stack([ssf[:N], ssf[NP:NP + N]], axis=0)
        if i < L - 1:
            h, t = _mlp_block(ss, dinv, eps_row, t, h, W1[i], b1[i], lng1[i],
                              lnb1[i], W2[i], b2[i], ln_g[i + 1], ln_b[i + 1])
        else:
            y = _mlp_block(ss, dinv, eps_row, t, h, W1[i], b1[i], lng1[i],
                           lnb1[i], W2[i], b2[i], gamma_out, beta_out,
                           W_out, b_out)
    return y


# speedup vs baseline: 5.2130x; 2.4534x over previous
import dataclasses
import functools

import jax
import jax.numpy as jnp
from jax import lax
from jax.experimental import pallas as pl
from jax.experimental.pallas import tpu as pltpu
from jax.experimental.pallas import tpu_sc as plsc

N = 10000
E = 160000
D_IN = 256
D_EDGE = 16
H = 256
OUT = 256
L = 3

BN = 1000
BE = 2000


def _ln(x, g, b, eps=1e-5):
    mu = jnp.mean(x, axis=-1, keepdims=True)
    var = jnp.mean((x - mu) ** 2, axis=-1, keepdims=True)
    return (x - mu) / jnp.sqrt(var + eps) * g + b


def _node_enc_kernel(x_ref, w_ref, b_ref, g_ref, bb_ref, h_ref, t_ref):
    h = jnp.dot(x_ref[...], w_ref[...], preferred_element_type=jnp.float32)
    h = h + b_ref[...]
    h_ref[...] = h
    t = jax.nn.relu(_ln(h, g_ref[...], bb_ref[...]))
    t_ref[0] = t[:, :128]
    t_ref[1] = t[:, 128:]


def _node_encoder(x, w, b, g, bb):
    nblk = N // BN
    return pl.pallas_call(
        _node_enc_kernel,
        grid=(nblk,),
        in_specs=[
            pl.BlockSpec((BN, D_IN), lambda i: (i, 0)),
            pl.BlockSpec((D_IN, H), lambda i: (0, 0)),
            pl.BlockSpec((1, H), lambda i: (0, 0)),
            pl.BlockSpec((1, H), lambda i: (0, 0)),
            pl.BlockSpec((1, H), lambda i: (0, 0)),
        ],
        out_specs=[
            pl.BlockSpec((BN, H), lambda i: (i, 0)),
            pl.BlockSpec((2, BN, 128), lambda i: (0, i, 0)),
        ],
        out_shape=[
            jax.ShapeDtypeStruct((N, H), jnp.float32),
            jax.ShapeDtypeStruct((2, N, 128), jnp.float32),
        ],
    )(x, w, b.reshape(1, H), g.reshape(1, H), bb.reshape(1, H))


def _edge_enc_kernel(a_ref, w_ref, b_ref, o_ref):
    ea = jnp.dot(a_ref[...], w_ref[...], preferred_element_type=jnp.float32)
    ea = ea + b_ref[...]
    o_ref[0] = ea[:, :128]
    o_ref[1] = ea[:, 128:]


def _edge_encoder(a, w, b):
    nblk = E // BE
    return pl.pallas_call(
        _edge_enc_kernel,
        grid=(nblk,),
        in_specs=[
            pl.BlockSpec((BE, D_EDGE), lambda i: (i, 0)),
            pl.BlockSpec((D_EDGE, H), lambda i: (0, 0)),
            pl.BlockSpec((1, H), lambda i: (0, 0)),
        ],
        out_specs=pl.BlockSpec((2, BE, 128), lambda i: (0, i, 0)),
        out_shape=jax.ShapeDtypeStruct((2, E, 128), jnp.float32),
    )(a, w, b.reshape(1, H))


def _mlp_mid_kernel(ss_ref, dinv_ref, eps_ref, t_ref, h_ref,
                    w1_ref, b1_ref, g1_ref, bb1_ref, w2_ref, b2_ref,
                    gn_ref, bn_ref, h_out, t_out):
    aggr = jnp.concatenate([ss_ref[0], ss_ref[1]], axis=-1)
    aggr = aggr * dinv_ref[...] + eps_ref[...]
    t = jnp.concatenate([t_ref[0], t_ref[1]], axis=-1)
    out = aggr + t
    m = jnp.dot(out, w1_ref[...], preferred_element_type=jnp.float32) + b1_ref[...]
    m = jax.nn.relu(_ln(m, g1_ref[...], bb1_ref[...]))
    m = jnp.dot(m, w2_ref[...], preferred_element_type=jnp.float32) + b2_ref[...]
    h_new = h_ref[...] + m
    h_out[...] = h_new
    tn = jax.nn.relu(_ln(h_new, gn_ref[...], bn_ref[...]))
    t_out[0] = tn[:, :128]
    t_out[1] = tn[:, 128:]


def _mlp_last_kernel(ss_ref, dinv_ref, eps_ref, t_ref, h_ref,
                     w1_ref, b1_ref, g1_ref, bb1_ref, w2_ref, b2_ref,
                     gn_ref, bn_ref, wo_ref, bo_ref, y_out):
    aggr = jnp.concatenate([ss_ref[0], ss_ref[1]], axis=-1)
    aggr = aggr * dinv_ref[...] + eps_ref[...]
    t = jnp.concatenate([t_ref[0], t_ref[1]], axis=-1)
    out = aggr + t
    m = jnp.dot(out, w1_ref[...], preferred_element_type=jnp.float32) + b1_ref[...]
    m = jax.nn.relu(_ln(m, g1_ref[...], bb1_ref[...]))
    m = jnp.dot(m, w2_ref[...], preferred_element_type=jnp.float32) + b2_ref[...]
    h_new = h_ref[...] + m
    y = jax.nn.relu(_ln(h_new, gn_ref[...], bn_ref[...]))
    y_out[...] = jnp.dot(y, wo_ref[...], preferred_element_type=jnp.float32) + bo_ref[...]


def _mlp_block(ss, dinv, eps_row, t, h, w1, b1, g1, bb1, w2, b2, gn, bn,
               wo=None, bo=None):
    nblk = N // BN
    in_specs = [
        pl.BlockSpec((2, BN, 128), lambda i: (0, i, 0)),
        pl.BlockSpec((BN, 1), lambda i: (i, 0)),
        pl.BlockSpec((BN, 1), lambda i: (i, 0)),
        pl.BlockSpec((2, BN, 128), lambda i: (0, i, 0)),
        pl.BlockSpec((BN, H), lambda i: (i, 0)),
        pl.BlockSpec((H, 2 * H), lambda i: (0, 0)),
        pl.BlockSpec((1, 2 * H), lambda i: (0, 0)),
        pl.BlockSpec((1, 2 * H), lambda i: (0, 0)),
        pl.BlockSpec((1, 2 * H), lambda i: (0, 0)),
        pl.BlockSpec((2 * H, H), lambda i: (0, 0)),
        pl.BlockSpec((1, H), lambda i: (0, 0)),
        pl.BlockSpec((1, H), lambda i: (0, 0)),
        pl.BlockSpec((1, H), lambda i: (0, 0)),
    ]
    args = [ss, dinv, eps_row, t, h, w1, b1.reshape(1, -1), g1.reshape(1, -1),
            bb1.reshape(1, -1), w2, b2.reshape(1, -1), gn.reshape(1, -1),
            bn.reshape(1, -1)]
    if wo is None:
        return pl.pallas_call(
            _mlp_mid_kernel,
            grid=(nblk,),
            in_specs=in_specs,
            out_specs=[
                pl.BlockSpec((BN, H), lambda i: (i, 0)),
                pl.BlockSpec((2, BN, 128), lambda i: (0, i, 0)),
            ],
            out_shape=[
                jax.ShapeDtypeStruct((N, H), jnp.float32),
                jax.ShapeDtypeStruct((2, N, 128), jnp.float32),
            ],
        )(*args)
    in_specs += [
        pl.BlockSpec((H, OUT), lambda i: (0, 0)),
        pl.BlockSpec((1, OUT), lambda i: (0, 0)),
    ]
    args += [wo, bo.reshape(1, OUT)]
    return pl.pallas_call(
        _mlp_last_kernel,
        grid=(nblk,),
        in_specs=in_specs,
        out_specs=pl.BlockSpec((BN, OUT), lambda i: (i, 0)),
        out_shape=jax.ShapeDtypeStruct((N, OUT), jnp.float32),
    )(*args)


_SC_MESH = plsc.VectorSubcoreMesh(core_axis_name="c", subcore_axis_name="s")
NSUB = 16
C = 80
EPW = E // NSUB
NCHUNK = EPW // C
NB = 4
NROW = E // C
NP = 10240

_SC_PARAMS = pltpu.CompilerParams()
if "needs_layout_passes" in pltpu.CompilerParams.__dataclass_fields__:
    _SC_PARAMS = dataclasses.replace(_SC_PARAMS, needs_layout_passes=False)


def _edge_sc_body(t_hbm, ea_hbm, srcadj_hbm, dst_hbm, out_hbm,
                  acc, rows, eab, sidx, didx,
                  sem_si, sem_di, sem_ea, sem_g, sem_sc):
    c = lax.axis_index("c")
    s = lax.axis_index("s")
    src_off = c * E + s * EPW
    dst_off = s * EPW
    ea_row0 = c * E + s * EPW
    acc_row0 = s * (NP // NSUB)

    @pl.loop(0, C)
    def _(r):
        for col in range(0, 128, 16):
            eab[0, r, pl.ds(col, 16)] = jnp.zeros((16,), jnp.float32)

    @pl.loop(0, (NP // NSUB) // C)
    def _(k):
        pltpu.sync_copy(eab.at[0], acc.at[pl.ds(acc_row0 + k * C, C)])

    plsc.subcore_barrier()

    def fetch(g):
        b = lax.rem(g, 2)
        bd = lax.rem(g, NB)
        pltpu.async_copy(srcadj_hbm.at[pl.ds(src_off + g * C, C)], sidx.at[b],
                         sem_si.at[b])
        pltpu.async_copy(dst_hbm.at[pl.ds(dst_off + g * C, C)], didx.at[bd],
                         sem_di.at[bd])
        pltpu.async_copy(ea_hbm.at[pl.ds(ea_row0 + g * C, C)], eab.at[b],
                         sem_ea.at[b])

    def drain_scatter(g):
        b = lax.rem(g, 2)
        bd = lax.rem(g, NB)
        pltpu.make_async_copy(rows.at[b], acc.at[didx.at[bd]],
                              sem_sc.at[b]).wait()

    def fire(g):
        b = lax.rem(g, 2)
        pltpu.make_async_copy(srcadj_hbm.at[pl.ds(src_off + g * C, C)],
                              sidx.at[b], sem_si.at[b]).wait()
        pltpu.async_copy(t_hbm.at[sidx.at[b]], rows.at[b], sem_g.at[b])

    def compute(g):
        b = lax.rem(g, 2)
        bd = lax.rem(g, NB)
        pltpu.make_async_copy(dst_hbm.at[pl.ds(dst_off + g * C, C)],
                              didx.at[bd], sem_di.at[bd]).wait()
        pltpu.make_async_copy(ea_hbm.at[pl.ds(ea_row0 + g * C, C)], eab.at[b],
                              sem_ea.at[b]).wait()
        pltpu.make_async_copy(t_hbm.at[sidx.at[b]], rows.at[b],
                              sem_g.at[b]).wait()

        @plsc.parallel_loop(0, C, unroll=4)
        def _(r):
            for col in range(0, 128, 16):
                v = rows[b, r, pl.ds(col, 16)] + eab[b, r, pl.ds(col, 16)]
                rows[b, r, pl.ds(col, 16)] = jnp.maximum(v, 0.0)

        pltpu.async_copy(rows.at[b], acc.at[didx.at[bd]], sem_sc.at[b],
                         add=True)

    fetch(0)
    fetch(1)
    fire(0)

    @pl.loop(0, NCHUNK)
    def _(g):
        @pl.when(g + 1 < NCHUNK)
        def _():
            @pl.when(g >= 2)
            def _():
                drain_scatter(g - 1)
            fire(g + 1)

        compute(g)

        @pl.when(g + 2 < NCHUNK)
        def _():
            fetch(g + 2)

    drain_scatter(NCHUNK - 2)
    drain_scatter(NCHUNK - 1)

    plsc.subcore_barrier()
    pltpu.sync_copy(acc.at[pl.ds(acc_row0, NP // NSUB)],
                    out_hbm.at[pl.ds(c * NP + acc_row0, NP // NSUB)])


@jax.jit
def _edge_pass_sc(t_flat, ea_flat, srcadj, dst1):
    f = pl.kernel(
        _edge_sc_body,
        out_type=jax.ShapeDtypeStruct((2 * NP, 128), jnp.float32),
        mesh=_SC_MESH,
        scratch_types=[
            pltpu.VMEM_SHARED((NP, 128), jnp.float32),
            pltpu.VMEM((2, C, 128), jnp.float32),
            pltpu.VMEM((2, C, 128), jnp.float32),
            pltpu.VMEM((2, C), jnp.int32),
            pltpu.VMEM((NB, C), jnp.int32),
            pltpu.SemaphoreType.DMA((2,)),
            pltpu.SemaphoreType.DMA((NB,)),
            pltpu.SemaphoreType.DMA((2,)),
            pltpu.SemaphoreType.DMA((2,)),
            pltpu.SemaphoreType.DMA((2,)),
        ],
        compiler_params=_SC_PARAMS,
    )
    return f(t_flat, ea_flat, srcadj, dst1)


NPAD = 10240
EDEG = E // 32


def _deg_sc_body(dst_hbm, out_hbm, hist, dbuf, sbuf, shared, sem):
    c = lax.axis_index("c")
    s = lax.axis_index("s")
    w = c * NSUB + s

    @pl.loop(0, NPAD, step=16)
    def _(i):
        hist[pl.ds(i, 16)] = jnp.zeros((16,), jnp.float32)

    pltpu.sync_copy(dst_hbm.at[pl.ds(w * EDEG, EDEG)], dbuf)
    ones = jnp.ones((16,), jnp.float32)

    @pl.loop(0, EDEG - 16, step=16)
    def _(i):
        plsc.addupdate_scatter(hist, [dbuf[pl.ds(i, 16)]], ones)

    tail_mask = lax.iota(jnp.int32, 16) >= 8
    plsc.addupdate_scatter(hist, [dbuf[pl.ds(EDEG - 16, 16)]], ones,
                           mask=tail_mask)

    pltpu.sync_copy(hist, shared.at[s])
    plsc.subcore_barrier()

    col0 = s * (NPAD // NSUB)
    pltpu.sync_copy(shared.at[:, pl.ds(col0, NPAD // NSUB)], sbuf)

    @pl.loop(0, NPAD // NSUB, step=16)
    def _(i):
        v = sbuf[0, pl.ds(i, 16)]
        for r in range(1, NSUB):
            v = v + sbuf[r, pl.ds(i, 16)]
        hist[pl.ds(i, 16)] = v

    pltpu.sync_copy(hist.at[pl.ds(0, NPAD // NSUB)],
                    out_hbm.at[c].at[pl.ds(col0, NPAD // NSUB)])


@jax.jit
def _degree_sc(dst):
    f = pl.kernel(
        _deg_sc_body,
        out_type=jax.ShapeDtypeStruct((2, NPAD), jnp.float32),
        mesh=_SC_MESH,
        scratch_types=[
            pltpu.VMEM((NPAD,), jnp.float32),
            pltpu.VMEM((EDEG,), jnp.int32),
            pltpu.VMEM((NSUB, NPAD // NSUB), jnp.float32),
            pltpu.VMEM_SHARED((NSUB, NPAD), jnp.float32),
            pltpu.SemaphoreType.DMA,
        ],
        compiler_params=_SC_PARAMS,
    )
    return f(dst)


def kernel(x, edge_index, edge_attr, W_node, b_node, W_edge, b_edge,
           ln_g, ln_b, W1, b1, lng1, lnb1, W2, b2,
           gamma_out, beta_out, W_out, b_out):
    src = edge_index[0]
    dst = edge_index[1]
    srcadj = jnp.concatenate([src, src + N], axis=0)

    h, t = _node_encoder(x, W_node, b_node, ln_g[0], ln_b[0])
    ea = _edge_encoder(edge_attr, W_edge, b_edge)
    ea_flat = ea.reshape(2 * E, 128)

    cntp = _degree_sc(dst)
    cnt = (cntp[0] + cntp[1])[:N]
    dinv = (1.0 / jnp.maximum(cnt, 1.0)).reshape(N, 1)
    eps_row = jnp.where(cnt > 0.0, jnp.float32(1e-7), 0.0).reshape(N, 1)

    for i in range(L):
        ssf = _edge_pass_sc(t.reshape(2 * N, 128), ea_flat, srcadj, dst)
        ss = jnp.stack([ssf[:N], ssf[NP:NP + N]], axis=0)
        if i < L - 1:
            h, t = _mlp_block(ss, dinv, eps_row, t, h, W1[i], b1[i], lng1[i],
                              lnb1[i], W2[i], b2[i], ln_g[i + 1], ln_b[i + 1])
        else:
            y = _mlp_block(ss, dinv, eps_row, t, h, W1[i], b1[i], lng1[i],
                           lnb1[i], W2[i], b2[i], gamma_out, beta_out,
                           W_out, b_out)
    return y


# speedup vs baseline: 5.3793x; 1.0319x over previous
import dataclasses
import functools

import jax
import jax.numpy as jnp
from jax import lax
from jax.experimental import pallas as pl
from jax.experimental.pallas import tpu as pltpu
from jax.experimental.pallas import tpu_sc as plsc

N = 10000
E = 160000
D_IN = 256
D_EDGE = 16
H = 256
OUT = 256
L = 3

BN = 1000
BE = 2000


def _pack_bf16(lo, hi):
    lo16 = jax.lax.bitcast_convert_type(lo.astype(jnp.bfloat16), jnp.uint16)
    hi16 = jax.lax.bitcast_convert_type(hi.astype(jnp.bfloat16), jnp.uint16)
    w = lo16.astype(jnp.uint32) | (hi16.astype(jnp.uint32) << 16)
    return jax.lax.bitcast_convert_type(w, jnp.float32)


def _ln(x, g, b, eps=1e-5):
    mu = jnp.mean(x, axis=-1, keepdims=True)
    var = jnp.mean((x - mu) ** 2, axis=-1, keepdims=True)
    return (x - mu) / jnp.sqrt(var + eps) * g + b


def _node_enc_kernel(x_ref, w_ref, b_ref, g_ref, bb_ref, h_ref, tf_ref, t_ref):
    h = jnp.dot(x_ref[...], w_ref[...], preferred_element_type=jnp.float32)
    h = h + b_ref[...]
    h_ref[...] = h
    t = jax.nn.relu(_ln(h, g_ref[...], bb_ref[...]))
    tf_ref[...] = t
    t_ref[...] = jnp.concatenate(
        [_pack_bf16(t[:, :64], t[:, 64:128]),
         _pack_bf16(t[:, 128:192], t[:, 192:])], axis=-1)


def _node_encoder(x, w, b, g, bb):
    nblk = N // BN
    return pl.pallas_call(
        _node_enc_kernel,
        grid=(nblk,),
        in_specs=[
            pl.BlockSpec((BN, D_IN), lambda i: (i, 0)),
            pl.BlockSpec((D_IN, H), lambda i: (0, 0)),
            pl.BlockSpec((1, H), lambda i: (0, 0)),
            pl.BlockSpec((1, H), lambda i: (0, 0)),
            pl.BlockSpec((1, H), lambda i: (0, 0)),
        ],
        out_specs=[
            pl.BlockSpec((BN, H), lambda i: (i, 0)),
            pl.BlockSpec((BN, H), lambda i: (i, 0)),
            pl.BlockSpec((BN, 128), lambda i: (i, 0)),
        ],
        out_shape=[
            jax.ShapeDtypeStruct((N, H), jnp.float32),
            jax.ShapeDtypeStruct((N, H), jnp.float32),
            jax.ShapeDtypeStruct((N, 128), jnp.float32),
        ],
    )(x, w, b.reshape(1, H), g.reshape(1, H), bb.reshape(1, H))


def _edge_enc_kernel(a_ref, w_ref, b_ref, o_ref):
    ea = jnp.dot(a_ref[...], w_ref[...], preferred_element_type=jnp.float32)
    ea = ea + b_ref[...]
    o_ref[0] = _pack_bf16(ea[:, :64], ea[:, 64:128])
    o_ref[1] = _pack_bf16(ea[:, 128:192], ea[:, 192:])


def _edge_encoder(a, w, b):
    nblk = E // BE
    return pl.pallas_call(
        _edge_enc_kernel,
        grid=(nblk,),
        in_specs=[
            pl.BlockSpec((BE, D_EDGE), lambda i: (i, 0)),
            pl.BlockSpec((D_EDGE, H), lambda i: (0, 0)),
            pl.BlockSpec((1, H), lambda i: (0, 0)),
        ],
        out_specs=pl.BlockSpec((2, BE, 64), lambda i: (0, i, 0)),
        out_shape=jax.ShapeDtypeStruct((2, E, 64), jnp.float32),
    )(a, w, b.reshape(1, H))


def _mlp_mid_kernel(ss_ref, dinv_ref, eps_ref, t_ref, h_ref,
                    w1_ref, b1_ref, g1_ref, bb1_ref, w2_ref, b2_ref,
                    gn_ref, bn_ref, h_out, tf_out, t_out):
    aggr = jnp.concatenate([ss_ref[0], ss_ref[1]], axis=-1)
    aggr = aggr * dinv_ref[...] + eps_ref[...]
    out = aggr + t_ref[...]
    m = jnp.dot(out, w1_ref[...], preferred_element_type=jnp.float32) + b1_ref[...]
    m = jax.nn.relu(_ln(m, g1_ref[...], bb1_ref[...]))
    m = jnp.dot(m, w2_ref[...], preferred_element_type=jnp.float32) + b2_ref[...]
    h_new = h_ref[...] + m
    h_out[...] = h_new
    tn = jax.nn.relu(_ln(h_new, gn_ref[...], bn_ref[...]))
    tf_out[...] = tn
    t_out[...] = jnp.concatenate(
        [_pack_bf16(tn[:, :64], tn[:, 64:128]),
         _pack_bf16(tn[:, 128:192], tn[:, 192:])], axis=-1)


def _mlp_last_kernel(ss_ref, dinv_ref, eps_ref, t_ref, h_ref,
                     w1_ref, b1_ref, g1_ref, bb1_ref, w2_ref, b2_ref,
                     gn_ref, bn_ref, wo_ref, bo_ref, y_out):
    aggr = jnp.concatenate([ss_ref[0], ss_ref[1]], axis=-1)
    aggr = aggr * dinv_ref[...] + eps_ref[...]
    out = aggr + t_ref[...]
    m = jnp.dot(out, w1_ref[...], preferred_element_type=jnp.float32) + b1_ref[...]
    m = jax.nn.relu(_ln(m, g1_ref[...], bb1_ref[...]))
    m = jnp.dot(m, w2_ref[...], preferred_element_type=jnp.float32) + b2_ref[...]
    h_new = h_ref[...] + m
    y = jax.nn.relu(_ln(h_new, gn_ref[...], bn_ref[...]))
    y_out[...] = jnp.dot(y, wo_ref[...], preferred_element_type=jnp.float32) + bo_ref[...]


def _mlp_block(ss, dinv, eps_row, t, h, w1, b1, g1, bb1, w2, b2, gn, bn,
               wo=None, bo=None):
    nblk = N // BN
    in_specs = [
        pl.BlockSpec((2, BN, 128), lambda i: (0, i, 0)),
        pl.BlockSpec((BN, 1), lambda i: (i, 0)),
        pl.BlockSpec((BN, 1), lambda i: (i, 0)),
        pl.BlockSpec((BN, H), lambda i: (i, 0)),
        pl.BlockSpec((BN, H), lambda i: (i, 0)),
        pl.BlockSpec((H, 2 * H), lambda i: (0, 0)),
        pl.BlockSpec((1, 2 * H), lambda i: (0, 0)),
        pl.BlockSpec((1, 2 * H), lambda i: (0, 0)),
        pl.BlockSpec((1, 2 * H), lambda i: (0, 0)),
        pl.BlockSpec((2 * H, H), lambda i: (0, 0)),
        pl.BlockSpec((1, H), lambda i: (0, 0)),
        pl.BlockSpec((1, H), lambda i: (0, 0)),
        pl.BlockSpec((1, H), lambda i: (0, 0)),
    ]
    args = [ss, dinv, eps_row, t, h, w1, b1.reshape(1, -1), g1.reshape(1, -1),
            bb1.reshape(1, -1), w2, b2.reshape(1, -1), gn.reshape(1, -1),
            bn.reshape(1, -1)]
    if wo is None:
        return pl.pallas_call(
            _mlp_mid_kernel,
            grid=(nblk,),
            in_specs=in_specs,
            out_specs=[
                pl.BlockSpec((BN, H), lambda i: (i, 0)),
                pl.BlockSpec((BN, H), lambda i: (i, 0)),
                pl.BlockSpec((BN, 128), lambda i: (i, 0)),
            ],
            out_shape=[
                jax.ShapeDtypeStruct((N, H), jnp.float32),
                jax.ShapeDtypeStruct((N, H), jnp.float32),
                jax.ShapeDtypeStruct((N, 128), jnp.float32),
            ],
        )(*args)
    in_specs += [
        pl.BlockSpec((H, OUT), lambda i: (0, 0)),
        pl.BlockSpec((1, OUT), lambda i: (0, 0)),
    ]
    args += [wo, bo.reshape(1, OUT)]
    return pl.pallas_call(
        _mlp_last_kernel,
        grid=(nblk,),
        in_specs=in_specs,
        out_specs=pl.BlockSpec((BN, OUT), lambda i: (i, 0)),
        out_shape=jax.ShapeDtypeStruct((N, OUT), jnp.float32),
    )(*args)


_SC_MESH = plsc.VectorSubcoreMesh(core_axis_name="c", subcore_axis_name="s")
NSUB = 16
C = 40
EPW = E // NSUB
NCHUNK = EPW // C
NSLOT = 4
NB = 8
NROW = E // C
NP = 10240

_SC_PARAMS = pltpu.CompilerParams()
if "needs_layout_passes" in pltpu.CompilerParams.__dataclass_fields__:
    _SC_PARAMS = dataclasses.replace(_SC_PARAMS, needs_layout_passes=False)


def _edge_sc_body(t_hbm, ea_hbm, src_hbm, dst_hbm, out_hbm,
                  acc, rows, eab, sidx, didx,
                  sem_si, sem_di, sem_ea, sem_g, sem_sc):
    c = lax.axis_index("c")
    s = lax.axis_index("s")
    e_off = s * EPW
    ea_row0 = c * E + s * EPW
    acc_row0 = s * (NP // NSUB)
    half = c * 64

    def fetch(g):
        b = lax.rem(g, NSLOT)
        bd = lax.rem(g, NB)
        pltpu.async_copy(src_hbm.at[pl.ds(e_off + g * C, C)], sidx.at[b],
                         sem_si.at[b])
        pltpu.async_copy(dst_hbm.at[pl.ds(e_off + g * C, C)], didx.at[bd],
                         sem_di.at[bd])
        pltpu.async_copy(ea_hbm.at[pl.ds(ea_row0 + g * C, C)], eab.at[b],
                         sem_ea.at[b])

    def drain_scatter(g):
        b = lax.rem(g, NSLOT)
        bd = lax.rem(g, NB)
        pltpu.make_async_copy(rows.at[b], acc.at[didx.at[bd]],
                              sem_sc.at[b]).wait()

    def fire(g):
        b = lax.rem(g, NSLOT)
        pltpu.make_async_copy(src_hbm.at[pl.ds(e_off + g * C, C)],
                              sidx.at[b], sem_si.at[b]).wait()
        pltpu.async_copy(t_hbm.at[sidx.at[b]], rows.at[b], sem_g.at[b])

    def compute(g):
        b = lax.rem(g, NSLOT)
        bd = lax.rem(g, NB)
        pltpu.make_async_copy(dst_hbm.at[pl.ds(e_off + g * C, C)],
                              didx.at[bd], sem_di.at[bd]).wait()
        pltpu.make_async_copy(ea_hbm.at[pl.ds(ea_row0 + g * C, C)], eab.at[b],
                              sem_ea.at[b]).wait()
        pltpu.make_async_copy(t_hbm.at[sidx.at[b]], rows.at[b],
                              sem_g.at[b]).wait()

        @plsc.parallel_loop(0, C, unroll=4)
        def _(r):
            for w in range(4):
                tv = plsc.bitcast(rows[b, r, pl.ds(half + w * 16, 16)],
                                  jnp.bfloat16)
                ev = plsc.bitcast(eab[b, r, pl.ds(w * 16, 16)], jnp.bfloat16)
                m = jnp.maximum(tv + ev, 0.0)
                a0, a1 = plsc.unpack(m, format=plsc.PackFormat.INTERLEAVED)
                rows[b, r, pl.ds(w * 16, 16)] = a0
                rows[b, r, pl.ds(64 + w * 16, 16)] = a1

        pltpu.async_copy(rows.at[b], acc.at[didx.at[bd]], sem_sc.at[b],
                         add=True)

    fetch(0)
    fetch(1)
    fetch(2)
    fetch(3)

    @pl.loop(0, C)
    def _(r):
        for col in range(0, 128, 16):
            rows[0, r, pl.ds(col, 16)] = jnp.zeros((16,), jnp.float32)

    @pl.loop(0, (NP // NSUB) // C)
    def _(k):
        pltpu.sync_copy(rows.at[0], acc.at[pl.ds(acc_row0 + k * C, C)])

    plsc.subcore_barrier()

    fire(0)
    fire(1)
    fire(2)

    @pl.loop(0, NCHUNK)
    def _(g):
        compute(g)

        @pl.when(g + 3 < NCHUNK)
        def _():
            @pl.when(g >= 1)
            def _():
                drain_scatter(g - 1)
            fire(g + 3)

        @pl.when(g + 4 < NCHUNK)
        def _():
            fetch(g + 4)

    drain_scatter(NCHUNK - 4)
    drain_scatter(NCHUNK - 3)
    drain_scatter(NCHUNK - 2)
    drain_scatter(NCHUNK - 1)

    plsc.subcore_barrier()
    pltpu.sync_copy(acc.at[pl.ds(acc_row0, NP // NSUB)],
                    out_hbm.at[pl.ds(c * NP + acc_row0, NP // NSUB)])


@jax.jit
def _edge_pass_sc(t_packed, ea_flat, src1, dst1):
    f = pl.kernel(
        _edge_sc_body,
        out_type=jax.ShapeDtypeStruct((2 * NP, 128), jnp.float32),
        mesh=_SC_MESH,
        scratch_types=[
            pltpu.VMEM_SHARED((NP, 128), jnp.float32),
            pltpu.VMEM((NSLOT, C, 128), jnp.float32),
            pltpu.VMEM((NSLOT, C, 64), jnp.float32),
            pltpu.VMEM((NSLOT, C), jnp.int32),
            pltpu.VMEM((NB, C), jnp.int32),
            pltpu.SemaphoreType.DMA((NSLOT,)),
            pltpu.SemaphoreType.DMA((NB,)),
            pltpu.SemaphoreType.DMA((NSLOT,)),
            pltpu.SemaphoreType.DMA((NSLOT,)),
            pltpu.SemaphoreType.DMA((NSLOT,)),
        ],
        compiler_params=_SC_PARAMS,
    )
    return f(t_packed, ea_flat, src1, dst1)


NPAD = 10240
EDEG = E // 32


def _deg_sc_body(dst_hbm, out_hbm, hist, dbuf, sbuf, shared, sem):
    c = lax.axis_index("c")
    s = lax.axis_index("s")
    w = c * NSUB + s

    @pl.loop(0, NPAD, step=16)
    def _(i):
        hist[pl.ds(i, 16)] = jnp.zeros((16,), jnp.float32)

    pltpu.sync_copy(dst_hbm.at[pl.ds(w * EDEG, EDEG)], dbuf)
    ones = jnp.ones((16,), jnp.float32)

    @pl.loop(0, EDEG - 16, step=16)
    def _(i):
        plsc.addupdate_scatter(hist, [dbuf[pl.ds(i, 16)]], ones)

    tail_mask = lax.iota(jnp.int32, 16) >= 8
    plsc.addupdate_scatter(hist, [dbuf[pl.ds(EDEG - 16, 16)]], ones,
                           mask=tail_mask)

    pltpu.sync_copy(hist, shared.at[s])
    plsc.subcore_barrier()

    col0 = s * (NPAD // NSUB)
    pltpu.sync_copy(shared.at[:, pl.ds(col0, NPAD // NSUB)], sbuf)

    @pl.loop(0, NPAD // NSUB, step=16)
    def _(i):
        v = sbuf[0, pl.ds(i, 16)]
        for r in range(1, NSUB):
            v = v + sbuf[r, pl.ds(i, 16)]
        hist[pl.ds(i, 16)] = v

    pltpu.sync_copy(hist.at[pl.ds(0, NPAD // NSUB)],
                    out_hbm.at[c].at[pl.ds(col0, NPAD // NSUB)])


@jax.jit
def _degree_sc(dst):
    f = pl.kernel(
        _deg_sc_body,
        out_type=jax.ShapeDtypeStruct((2, NPAD), jnp.float32),
        mesh=_SC_MESH,
        scratch_types=[
            pltpu.VMEM((NPAD,), jnp.float32),
            pltpu.VMEM((EDEG,), jnp.int32),
            pltpu.VMEM((NSUB, NPAD // NSUB), jnp.float32),
            pltpu.VMEM_SHARED((NSUB, NPAD), jnp.float32),
            pltpu.SemaphoreType.DMA,
        ],
        compiler_params=_SC_PARAMS,
    )
    return f(dst)


def kernel(x, edge_index, edge_attr, W_node, b_node, W_edge, b_edge,
           ln_g, ln_b, W1, b1, lng1, lnb1, W2, b2,
           gamma_out, beta_out, W_out, b_out):
    src = edge_index[0]
    dst = edge_index[1]

    h, tf, tsc = _node_encoder(x, W_node, b_node, ln_g[0], ln_b[0])
    ea = _edge_encoder(edge_attr, W_edge, b_edge)
    ea_flat = ea.reshape(2 * E, 64)

    cntp = _degree_sc(dst)
    cnt = (cntp[0] + cntp[1])[:N]
    dinv = (1.0 / jnp.maximum(cnt, 1.0)).reshape(N, 1)
    eps_row = jnp.where(cnt > 0.0, jnp.float32(1e-7), 0.0).reshape(N, 1)

    for i in range(L):
        ssf = _edge_pass_sc(tsc, ea_flat, src, dst)
        ss = jnp.stack([ssf[:N], ssf[NP:NP + N]], axis=0)
        if i < L - 1:
            h, tf, tsc = _mlp_block(ss, dinv, eps_row, tf, h, W1[i], b1[i],
                                    lng1[i], lnb1[i], W2[i], b2[i],
                                    ln_g[i + 1], ln_b[i + 1])
        else:
            y = _mlp_block(ss, dinv, eps_row, tf, h, W1[i], b1[i], lng1[i],
                           lnb1[i], W2[i], b2[i], gamma_out, beta_out,
                           W_out, b_out)
    return y


# speedup vs baseline: 5.4873x; 1.0201x over previous
import dataclasses
import functools

import jax
import jax.numpy as jnp
from jax import lax
from jax.experimental import pallas as pl
from jax.experimental.pallas import tpu as pltpu
from jax.experimental.pallas import tpu_sc as plsc

N = 10000
E = 160000
D_IN = 256
D_EDGE = 16
H = 256
OUT = 256
L = 3

BN = 1024
BE = 2000


def _pack_bf16(lo, hi):
    lo16 = jax.lax.bitcast_convert_type(lo.astype(jnp.bfloat16), jnp.uint16)
    hi16 = jax.lax.bitcast_convert_type(hi.astype(jnp.bfloat16), jnp.uint16)
    w = lo16.astype(jnp.uint32) | (hi16.astype(jnp.uint32) << 16)
    return jax.lax.bitcast_convert_type(w, jnp.float32)


def _ln(x, g, b, eps=1e-5):
    mu = jnp.mean(x, axis=-1, keepdims=True)
    var = jnp.mean((x - mu) ** 2, axis=-1, keepdims=True)
    return (x - mu) / jnp.sqrt(var + eps) * g + b


def _node_enc_kernel(x_ref, w_ref, b_ref, g_ref, bb_ref, h_ref, tf_ref, t_ref):
    h = jnp.dot(x_ref[...], w_ref[...], preferred_element_type=jnp.float32)
    h = h + b_ref[...]
    h_ref[...] = h
    t = jax.nn.relu(_ln(h, g_ref[...], bb_ref[...]))
    tf_ref[...] = t
    t_ref[...] = jnp.concatenate(
        [_pack_bf16(t[:, :64], t[:, 64:128]),
         _pack_bf16(t[:, 128:192], t[:, 192:])], axis=-1)


def _node_encoder(x, w, b, g, bb):
    nblk = NP // BN
    return pl.pallas_call(
        _node_enc_kernel,
        grid=(nblk,),
        in_specs=[
            pl.BlockSpec((BN, D_IN), lambda i: (i, 0)),
            pl.BlockSpec((D_IN, H), lambda i: (0, 0)),
            pl.BlockSpec((1, H), lambda i: (0, 0)),
            pl.BlockSpec((1, H), lambda i: (0, 0)),
            pl.BlockSpec((1, H), lambda i: (0, 0)),
        ],
        out_specs=[
            pl.BlockSpec((BN, H), lambda i: (i, 0)),
            pl.BlockSpec((BN, H), lambda i: (i, 0)),
            pl.BlockSpec((BN, 128), lambda i: (i, 0)),
        ],
        out_shape=[
            jax.ShapeDtypeStruct((NP, H), jnp.float32),
            jax.ShapeDtypeStruct((NP, H), jnp.float32),
            jax.ShapeDtypeStruct((NP, 128), jnp.float32),
        ],
    )(x, w, b.reshape(1, H), g.reshape(1, H), bb.reshape(1, H))


def _edge_enc_kernel(a_ref, w_ref, b_ref, o_ref):
    ea = jnp.dot(a_ref[...], w_ref[...], preferred_element_type=jnp.float32)
    ea = ea + b_ref[...]
    o_ref[0] = _pack_bf16(ea[:, :64], ea[:, 64:128])
    o_ref[1] = _pack_bf16(ea[:, 128:192], ea[:, 192:])


def _edge_encoder(a, w, b):
    nblk = E // BE
    return pl.pallas_call(
        _edge_enc_kernel,
        grid=(nblk,),
        in_specs=[
            pl.BlockSpec((BE, D_EDGE), lambda i: (i, 0)),
            pl.BlockSpec((D_EDGE, H), lambda i: (0, 0)),
            pl.BlockSpec((1, H), lambda i: (0, 0)),
        ],
        out_specs=pl.BlockSpec((2, BE, 64), lambda i: (0, i, 0)),
        out_shape=jax.ShapeDtypeStruct((2, E, 64), jnp.float32),
    )(a, w, b.reshape(1, H))


def _mlp_mid_kernel(ssa_ref, ssb_ref, dinv_ref, eps_ref, t_ref, h_ref,
                    w1_ref, b1_ref, g1_ref, bb1_ref, w2_ref, b2_ref,
                    gn_ref, bn_ref, h_out, tf_out, t_out):
    aggr = jnp.concatenate([ssa_ref[...], ssb_ref[...]], axis=-1)
    aggr = aggr * dinv_ref[...] + eps_ref[...]
    out = aggr + t_ref[...]
    m = jnp.dot(out, w1_ref[...], preferred_element_type=jnp.float32) + b1_ref[...]
    m = jax.nn.relu(_ln(m, g1_ref[...], bb1_ref[...]))
    m = jnp.dot(m, w2_ref[...], preferred_element_type=jnp.float32) + b2_ref[...]
    h_new = h_ref[...] + m
    h_out[...] = h_new
    tn = jax.nn.relu(_ln(h_new, gn_ref[...], bn_ref[...]))
    tf_out[...] = tn
    t_out[...] = jnp.concatenate(
        [_pack_bf16(tn[:, :64], tn[:, 64:128]),
         _pack_bf16(tn[:, 128:192], tn[:, 192:])], axis=-1)


def _mlp_last_kernel(ssa_ref, ssb_ref, dinv_ref, eps_ref, t_ref, h_ref,
                     w1_ref, b1_ref, g1_ref, bb1_ref, w2_ref, b2_ref,
                     gn_ref, bn_ref, wo_ref, bo_ref, y_out):
    aggr = jnp.concatenate([ssa_ref[...], ssb_ref[...]], axis=-1)
    aggr = aggr * dinv_ref[...] + eps_ref[...]
    out = aggr + t_ref[...]
    m = jnp.dot(out, w1_ref[...], preferred_element_type=jnp.float32) + b1_ref[...]
    m = jax.nn.relu(_ln(m, g1_ref[...], bb1_ref[...]))
    m = jnp.dot(m, w2_ref[...], preferred_element_type=jnp.float32) + b2_ref[...]
    h_new = h_ref[...] + m
    y = jax.nn.relu(_ln(h_new, gn_ref[...], bn_ref[...]))
    y_out[...] = jnp.dot(y, wo_ref[...], preferred_element_type=jnp.float32) + bo_ref[...]


def _mlp_block(ssf, dinv, eps_row, t, h, w1, b1, g1, bb1, w2, b2, gn, bn,
               wo=None, bo=None):
    nblk = NP // BN
    in_specs = [
        pl.BlockSpec((BN, 128), lambda i: (i, 0)),
        pl.BlockSpec((BN, 128), lambda i: (NP // BN + i, 0)),
        pl.BlockSpec((BN, 1), lambda i: (i, 0)),
        pl.BlockSpec((BN, 1), lambda i: (i, 0)),
        pl.BlockSpec((BN, H), lambda i: (i, 0)),
        pl.BlockSpec((BN, H), lambda i: (i, 0)),
        pl.BlockSpec((H, 2 * H), lambda i: (0, 0)),
        pl.BlockSpec((1, 2 * H), lambda i: (0, 0)),
        pl.BlockSpec((1, 2 * H), lambda i: (0, 0)),
        pl.BlockSpec((1, 2 * H), lambda i: (0, 0)),
        pl.BlockSpec((2 * H, H), lambda i: (0, 0)),
        pl.BlockSpec((1, H), lambda i: (0, 0)),
        pl.BlockSpec((1, H), lambda i: (0, 0)),
        pl.BlockSpec((1, H), lambda i: (0, 0)),
    ]
    args = [ssf, ssf, dinv, eps_row, t, h, w1, b1.reshape(1, -1), g1.reshape(1, -1),
            bb1.reshape(1, -1), w2, b2.reshape(1, -1), gn.reshape(1, -1),
            bn.reshape(1, -1)]
    if wo is None:
        return pl.pallas_call(
            _mlp_mid_kernel,
            grid=(nblk,),
            in_specs=in_specs,
            out_specs=[
                pl.BlockSpec((BN, H), lambda i: (i, 0)),
                pl.BlockSpec((BN, H), lambda i: (i, 0)),
                pl.BlockSpec((BN, 128), lambda i: (i, 0)),
            ],
            out_shape=[
                jax.ShapeDtypeStruct((NP, H), jnp.float32),
                jax.ShapeDtypeStruct((NP, H), jnp.float32),
                jax.ShapeDtypeStruct((NP, 128), jnp.float32),
            ],
        )(*args)
    in_specs += [
        pl.BlockSpec((H, OUT), lambda i: (0, 0)),
        pl.BlockSpec((1, OUT), lambda i: (0, 0)),
    ]
    args += [wo, bo.reshape(1, OUT)]
    return pl.pallas_call(
        _mlp_last_kernel,
        grid=(nblk,),
        in_specs=in_specs,
        out_specs=pl.BlockSpec((BN, OUT), lambda i: (i, 0)),
        out_shape=jax.ShapeDtypeStruct((NP, OUT), jnp.float32),
    )(*args)


_SC_MESH = plsc.VectorSubcoreMesh(core_axis_name="c", subcore_axis_name="s")
NSUB = 16
C = 40
EPW = E // NSUB
NCHUNK = EPW // C
NSLOT = 4
NB = 8
NROW = E // C
NP = 10240

_SC_PARAMS = pltpu.CompilerParams()
if "needs_layout_passes" in pltpu.CompilerParams.__dataclass_fields__:
    _SC_PARAMS = dataclasses.replace(_SC_PARAMS, needs_layout_passes=False)


def _edge_sc_body(t_hbm, ea_hbm, src_hbm, dst_hbm, out_hbm,
                  acc, rows, eab, sidx, didx,
                  sem_si, sem_di, sem_ea, sem_g, sem_sc):
    c = lax.axis_index("c")
    s = lax.axis_index("s")
    e_off = s * EPW
    ea_row0 = c * E + s * EPW
    acc_row0 = s * (NP // NSUB)
    half = c * 64

    @pl.loop(0, C)
    def _(r):
        for col in range(0, 128, 16):
            rows[0, r, pl.ds(col, 16)] = jnp.zeros((16,), jnp.float32)

    @pl.loop(0, (NP // NSUB) // C)
    def _(k):
        pltpu.sync_copy(rows.at[0], acc.at[pl.ds(acc_row0 + k * C, C)])

    plsc.subcore_barrier()

    def fetch(g):
        b = lax.rem(g, NSLOT)
        bd = lax.rem(g, NB)
        pltpu.async_copy(src_hbm.at[pl.ds(e_off + g * C, C)], sidx.at[b],
                         sem_si.at[b])
        pltpu.async_copy(dst_hbm.at[pl.ds(e_off + g * C, C)], didx.at[bd],
                         sem_di.at[bd])
        pltpu.async_copy(ea_hbm.at[pl.ds(ea_row0 + g * C, C)], eab.at[b],
                         sem_ea.at[b])

    def drain_scatter(g):
        b = lax.rem(g, NSLOT)
        bd = lax.rem(g, NB)
        pltpu.make_async_copy(rows.at[b], acc.at[didx.at[bd]],
                              sem_sc.at[b]).wait()

    def fire(g):
        b = lax.rem(g, NSLOT)
        pltpu.make_async_copy(src_hbm.at[pl.ds(e_off + g * C, C)],
                              sidx.at[b], sem_si.at[b]).wait()
        pltpu.async_copy(t_hbm.at[sidx.at[b]], rows.at[b], sem_g.at[b])

    def compute(g):
        b = lax.rem(g, NSLOT)
        bd = lax.rem(g, NB)
        pltpu.make_async_copy(dst_hbm.at[pl.ds(e_off + g * C, C)],
                              didx.at[bd], sem_di.at[bd]).wait()
        pltpu.make_async_copy(ea_hbm.at[pl.ds(ea_row0 + g * C, C)], eab.at[b],
                              sem_ea.at[b]).wait()
        pltpu.make_async_copy(t_hbm.at[sidx.at[b]], rows.at[b],
                              sem_g.at[b]).wait()

        @plsc.parallel_loop(0, C, unroll=4)
        def _(r):
            for w in range(4):
                tv = plsc.bitcast(rows[b, r, pl.ds(half + w * 16, 16)],
                                  jnp.bfloat16)
                ev = plsc.bitcast(eab[b, r, pl.ds(w * 16, 16)], jnp.bfloat16)
                m = jnp.maximum(tv + ev, 0.0)
                a0, a1 = plsc.unpack(m, format=plsc.PackFormat.INTERLEAVED)
                rows[b, r, pl.ds(w * 16, 16)] = a0
                rows[b, r, pl.ds(64 + w * 16, 16)] = a1

        pltpu.async_copy(rows.at[b], acc.at[didx.at[bd]], sem_sc.at[b],
                         add=True)

    fetch(0)
    fetch(1)
    fetch(2)
    fetch(3)
    fire(0)
    fire(1)

    @pl.loop(0, NCHUNK)
    def _(g):
        @pl.when(g + 2 < NCHUNK)
        def _():
            @pl.when(g >= 2)
            def _():
                drain_scatter(g - 2)
            fire(g + 2)

        compute(g)

        @pl.when(g + 4 < NCHUNK)
        def _():
            fetch(g + 4)

    drain_scatter(NCHUNK - 4)
    drain_scatter(NCHUNK - 3)
    drain_scatter(NCHUNK - 2)
    drain_scatter(NCHUNK - 1)

    plsc.subcore_barrier()
    pltpu.sync_copy(acc.at[pl.ds(acc_row0, NP // NSUB)],
                    out_hbm.at[pl.ds(c * NP + acc_row0, NP // NSUB)])


@jax.jit
def _edge_pass_sc(t_packed, ea_flat, src1, dst1):
    f = pl.kernel(
        _edge_sc_body,
        out_type=jax.ShapeDtypeStruct((2 * NP, 128), jnp.float32),
        mesh=_SC_MESH,
        scratch_types=[
            pltpu.VMEM_SHARED((NP, 128), jnp.float32),
            pltpu.VMEM((NSLOT, C, 128), jnp.float32),
            pltpu.VMEM((NSLOT, C, 64), jnp.float32),
            pltpu.VMEM((NSLOT, C), jnp.int32),
            pltpu.VMEM((NB, C), jnp.int32),
            pltpu.SemaphoreType.DMA((NSLOT,)),
            pltpu.SemaphoreType.DMA((NB,)),
            pltpu.SemaphoreType.DMA((NSLOT,)),
            pltpu.SemaphoreType.DMA((NSLOT,)),
            pltpu.SemaphoreType.DMA((NSLOT,)),
        ],
        compiler_params=_SC_PARAMS,
    )
    return f(t_packed, ea_flat, src1, dst1)


NPAD = 10240
EDEG = E // 32


def _deg_sc_body(dst_hbm, out_hbm, hist, dbuf, sbuf, shared, sem):
    c = lax.axis_index("c")
    s = lax.axis_index("s")
    w = c * NSUB + s

    @pl.loop(0, NPAD, step=16)
    def _(i):
        hist[pl.ds(i, 16)] = jnp.zeros((16,), jnp.float32)

    pltpu.sync_copy(dst_hbm.at[pl.ds(w * EDEG, EDEG)], dbuf)
    ones = jnp.ones((16,), jnp.float32)

    @pl.loop(0, EDEG - 16, step=16)
    def _(i):
        plsc.addupdate_scatter(hist, [dbuf[pl.ds(i, 16)]], ones)

    tail_mask = lax.iota(jnp.int32, 16) >= 8
    plsc.addupdate_scatter(hist, [dbuf[pl.ds(EDEG - 16, 16)]], ones,
                           mask=tail_mask)

    pltpu.sync_copy(hist, shared.at[s])
    plsc.subcore_barrier()

    col0 = s * (NPAD // NSUB)
    pltpu.sync_copy(shared.at[:, pl.ds(col0, NPAD // NSUB)], sbuf)

    @pl.loop(0, NPAD // NSUB, step=16)
    def _(i):
        v = sbuf[0, pl.ds(i, 16)]
        for r in range(1, NSUB):
            v = v + sbuf[r, pl.ds(i, 16)]
        hist[pl.ds(i, 16)] = v

    pltpu.sync_copy(hist.at[pl.ds(0, NPAD // NSUB)],
                    out_hbm.at[c].at[pl.ds(col0, NPAD // NSUB)])


@jax.jit
def _degree_sc(dst):
    f = pl.kernel(
        _deg_sc_body,
        out_type=jax.ShapeDtypeStruct((2, NPAD), jnp.float32),
        mesh=_SC_MESH,
        scratch_types=[
            pltpu.VMEM((NPAD,), jnp.float32),
            pltpu.VMEM((EDEG,), jnp.int32),
            pltpu.VMEM((NSUB, NPAD // NSUB), jnp.float32),
            pltpu.VMEM_SHARED((NSUB, NPAD), jnp.float32),
            pltpu.SemaphoreType.DMA,
        ],
        compiler_params=_SC_PARAMS,
    )
    return f(dst)


def kernel(x, edge_index, edge_attr, W_node, b_node, W_edge, b_edge,
           ln_g, ln_b, W1, b1, lng1, lnb1, W2, b2,
           gamma_out, beta_out, W_out, b_out):
    src = edge_index[0]
    dst = edge_index[1]

    x_pad = jnp.pad(x, ((0, NP - N), (0, 0)))
    h, tf, tsc = _node_encoder(x_pad, W_node, b_node, ln_g[0], ln_b[0])
    ea = _edge_encoder(edge_attr, W_edge, b_edge)
    ea_flat = ea.reshape(2 * E, 64)

    cntp = _degree_sc(dst)
    cnt = cntp[0] + cntp[1]
    dinv = (1.0 / jnp.maximum(cnt, 1.0)).reshape(NP, 1)
    eps_row = jnp.where(cnt > 0.0, jnp.float32(1e-7), 0.0).reshape(NP, 1)

    for i in range(L):
        ssf = _edge_pass_sc(tsc, ea_flat, src, dst)
        if i < L - 1:
            h, tf, tsc = _mlp_block(ssf, dinv, eps_row, tf, h, W1[i], b1[i],
                                    lng1[i], lnb1[i], W2[i], b2[i],
                                    ln_g[i + 1], ln_b[i + 1])
        else:
            y = _mlp_block(ssf, dinv, eps_row, tf, h, W1[i], b1[i], lng1[i],
                           lnb1[i], W2[i], b2[i], gamma_out, beta_out,
                           W_out, b_out)
    return y[:N]
